# baseline (speedup 1.0000x reference)
import functools
import math

import jax
import jax.numpy as jnp
from jax import lax
from jax.experimental import pallas as pl
from jax.experimental.pallas import tpu as pltpu
from jax.experimental.pallas import tpu_sc as plsc

N = 10000
E = 160000
DC = 16
HID = 32

NC = 2
NS = 16
NW = NC * NS
LANES = 16

NP = 10240
CHUNK = E // NW
FULL = CHUNK // LANES
TAIL = CHUNK - FULL * LANES
CCAP = (FULL + 1) * LANES


def _sc_scatter(orc, ei_flat):
    mesh = plsc.VectorSubcoreMesh(core_axis_name="c", subcore_axis_name="s")

    @functools.partial(
        pl.kernel,
        out_type=(
            jax.ShapeDtypeStruct((NW, NP), jnp.float32),
            jax.ShapeDtypeStruct((NW, NP), jnp.float32),
        ),
        mesh=mesh,
        compiler_params=pltpu.CompilerParams(
            use_tc_tiling_on_sc=False, needs_layout_passes=False),
        scratch_types=(
            pltpu.VMEM((NP,), jnp.float32),
            pltpu.VMEM((CCAP,), jnp.int32),
            pltpu.VMEM((CCAP,), jnp.int32),
            pltpu.VMEM((NP,), jnp.float32),
            pltpu.VMEM((NP,), jnp.float32),
            pltpu.SemaphoreType.DMA,
            pltpu.SemaphoreType.DMA,
            pltpu.SemaphoreType.DMA,
        ),
    )
    def sc_kernel(orc_hbm, ei_hbm, msum_hbm, cnt_hbm,
                  orc_v, u_v, v_v, ms_v, cn_v, sem0, sem1, sem2):
        wid = lax.axis_index("s") * NC + lax.axis_index("c")
        base = wid * CHUNK
        junk = jnp.full((LANES,), N, jnp.int32)
        u_v[pl.ds(FULL * LANES, LANES)] = junk
        v_v[pl.ds(FULL * LANES, LANES)] = junk
        cp0 = pltpu.async_copy(orc_hbm, orc_v.at[pl.ds(0, N)], sem0)
        cp1 = pltpu.async_copy(ei_hbm.at[0, pl.ds(base, CHUNK)],
                               u_v.at[pl.ds(0, CHUNK)], sem1)
        cp2 = pltpu.async_copy(ei_hbm.at[1, pl.ds(base, CHUNK)],
                               v_v.at[pl.ds(0, CHUNK)], sem2)

        zeros = jnp.zeros((LANES,), jnp.float32)
        ZUNROLL = 8

        def zero_body(i, carry):
            for k in range(ZUNROLL):
                off = (i * ZUNROLL + k) * LANES
                ms_v[pl.ds(off, LANES)] = zeros
                cn_v[pl.ds(off, LANES)] = zeros
            return carry

        lax.fori_loop(0, NP // (LANES * ZUNROLL), zero_body, 0)
        cp0.wait()
        cp1.wait()
        cp2.wait()

        ones = jnp.ones((LANES,), jnp.float32)

        def step(off):
            u16 = u_v[pl.ds(off, LANES)]
            v16 = v_v[pl.ds(off, LANES)]
            ou = plsc.load_gather(orc_v, [u16])
            ov = plsc.load_gather(orc_v, [v16])
            plsc.addupdate_scatter(ms_v, [u16], ov)
            plsc.addupdate_scatter(ms_v, [v16], ou)
            plsc.addupdate_scatter(cn_v, [u16], ones)
            plsc.addupdate_scatter(cn_v, [v16], ones)

        UNROLL = 8

        def body(i, carry):
            for k in range(UNROLL):
                step(i * (UNROLL * LANES) + k * LANES)
            return carry

        lax.fori_loop(0, FULL // UNROLL, body, 0)
        for k in range(FULL % UNROLL + 1):
            step((FULL // UNROLL * UNROLL + k) * LANES)

        st0 = pltpu.async_copy(ms_v, msum_hbm.at[wid], sem0)
        st1 = pltpu.async_copy(cn_v, cnt_hbm.at[wid], sem1)
        st0.wait()
        st1.wait()

    return sc_kernel(orc, ei_flat)


def _tc_body(orc_ref, ms_ref, cn_ref, w1_ref, b1_ref, w2_ref, b2_ref,
             g_ref, be_ref, out_ref):
    f32 = jnp.float32
    msum = jnp.sum(ms_ref[...], axis=0, keepdims=True)
    cnt = jnp.sum(cn_ref[...], axis=0, keepdims=True)
    nm = jnp.where(cnt > 0, msum / jnp.where(cnt > 0, cnt, 1.0), 0.0)
    orc = orc_ref[...]
    scale = 1.0 / (2.0 + 1e-8)
    no = jnp.clip((orc + 1.0) * scale, 0.0, 1.0)
    nn = jnp.clip((nm + 1.0) * scale, 0.0, 1.0)

    j = lax.broadcasted_iota(jnp.int32, (DC, 1), 0)
    base = jnp.where(j < DC // 2, no, nn)
    freq = (((j % (DC // 2)) // 2) + 1).astype(f32) * math.pi
    ang = base * freq
    phi = jnp.where(j % 2 == 0, jnp.sin(ang), jnp.cos(ang))

    h = lax.dot_general(w1_ref[...], phi, (((1,), (0,)), ((), ())),
                        preferred_element_type=f32) + b1_ref[...]
    h = jnp.maximum(h, 0.0)
    h2 = lax.dot_general(w2_ref[...], h, (((1,), (0,)), ((), ())),
                         preferred_element_type=f32) + b2_ref[...]
    mu = jnp.mean(h2, axis=0, keepdims=True)
    d = h2 - mu
    var = jnp.mean(d * d, axis=0, keepdims=True)
    ln = d / jnp.sqrt(var + 1e-5) * g_ref[...] + be_ref[...]
    outT = ln + phi
    eye = (lax.broadcasted_iota(jnp.int32, (DC, DC), 0)
           == lax.broadcasted_iota(jnp.int32, (DC, DC), 1)).astype(f32)
    out_ref[...] = lax.dot_general(outT, eye, (((0,), (0,)), ((), ())),
                                   preferred_element_type=f32)


def _tc_dense(orc_row, msum_p, cnt_p, W1, b1, W2, b2, gamma, beta,
              interpret=False):
    C = 2048
    return pl.pallas_call(
        _tc_body,
        grid=(NP // C,),
        in_specs=[
            pl.BlockSpec((1, C), lambda i: (0, i)),
            pl.BlockSpec((NW, C), lambda i: (0, i)),
            pl.BlockSpec((NW, C), lambda i: (0, i)),
            pl.BlockSpec((HID, DC), lambda i: (0, 0)),
            pl.BlockSpec((HID, 1), lambda i: (0, 0)),
            pl.BlockSpec((DC, HID), lambda i: (0, 0)),
            pl.BlockSpec((DC, 1), lambda i: (0, 0)),
            pl.BlockSpec((DC, 1), lambda i: (0, 0)),
            pl.BlockSpec((DC, 1), lambda i: (0, 0)),
        ],
        out_specs=pl.BlockSpec((C, DC), lambda i: (i, 0)),
        out_shape=jax.ShapeDtypeStruct((NP, DC), jnp.float32),
        interpret=interpret,
    )(orc_row, msum_p, cnt_p, W1, b1, W2, b2, gamma, beta)


def kernel(node_orc, edge_index, W1, b1, W2, b2, gamma, beta):
    msum_p, cnt_p = _sc_scatter(node_orc, edge_index)
    out = _tc_dense(node_orc.reshape(1, N), msum_p, cnt_p,
                    W1, b1.reshape(HID, 1), W2, b2.reshape(DC, 1),
                    gamma.reshape(DC, 1), beta.reshape(DC, 1))
    return out[:N]


# speedup vs baseline: 1.0330x; 1.0330x over previous
import functools
import math

import jax
import jax.numpy as jnp
from jax import lax
from jax.experimental import pallas as pl
from jax.experimental.pallas import tpu as pltpu
from jax.experimental.pallas import tpu_sc as plsc

N = 10000
E = 160000
DC = 16
HID = 32

NC = 2
NS = 16
NW = NC * NS
LANES = 16

NP = 10240
CBIG = 5120
CSML = 4992


def _sc_scatter(orc_p, ei):
    mesh = plsc.VectorSubcoreMesh(core_axis_name="c", subcore_axis_name="s")

    @functools.partial(
        pl.kernel,
        out_type=(
            jax.ShapeDtypeStruct((NW * NP,), jnp.float32),
            jax.ShapeDtypeStruct((NW * NP,), jnp.float32),
        ),
        mesh=mesh,
        compiler_params=pltpu.CompilerParams(
            use_tc_tiling_on_sc=True, needs_layout_passes=False),
        scratch_types=(
            pltpu.VMEM((NP,), jnp.float32),
            pltpu.VMEM((2, CBIG), jnp.int32),
            pltpu.VMEM((NP,), jnp.float32),
            pltpu.VMEM((NP,), jnp.float32),
            pltpu.SemaphoreType.DMA,
            pltpu.SemaphoreType.DMA,
        ),
    )
    def sc_kernel(orc_hbm, ei_hbm, msum_hbm, cnt_hbm,
                  orc_v, uv_v, ms_v, cn_v, sem0, sem1):
        wid = lax.axis_index("s") * NC + lax.axis_index("c")
        big = wid < 2
        base = pl.multiple_of(
            jnp.where(big, wid * CBIG, 2 * CBIG + (wid - 2) * CSML), 128)
        cp0 = pltpu.async_copy(orc_hbm, orc_v, sem0)

        @pl.when(big)
        def _():
            cp1 = pltpu.async_copy(ei_hbm.at[:, pl.ds(base, CBIG)],
                                   uv_v.at[:, pl.ds(0, CBIG)], sem1)
            cp1.wait()

        @pl.when(jnp.logical_not(big))
        def _():
            cp1 = pltpu.async_copy(ei_hbm.at[:, pl.ds(base, CSML)],
                                   uv_v.at[:, pl.ds(0, CSML)], sem1)
            cp1.wait()

        zeros = jnp.zeros((LANES,), jnp.float32)
        ZUNROLL = 8

        def zero_body(i, carry):
            for k in range(ZUNROLL):
                off = (i * ZUNROLL + k) * LANES
                ms_v[pl.ds(off, LANES)] = zeros
                cn_v[pl.ds(off, LANES)] = zeros
            return carry

        lax.fori_loop(0, NP // (LANES * ZUNROLL), zero_body, 0)
        cp0.wait()

        ones = jnp.ones((LANES,), jnp.float32)

        def step(off):
            u16 = uv_v[0, pl.ds(off, LANES)]
            v16 = uv_v[1, pl.ds(off, LANES)]
            ou = plsc.load_gather(orc_v, [u16])
            ov = plsc.load_gather(orc_v, [v16])
            plsc.addupdate_scatter(ms_v, [u16], ov)
            plsc.addupdate_scatter(ms_v, [v16], ou)
            plsc.addupdate_scatter(cn_v, [u16], ones)
            plsc.addupdate_scatter(cn_v, [v16], ones)

        UNROLL = 8
        nsteps = jnp.where(big, CBIG // (UNROLL * LANES),
                           CSML // (UNROLL * LANES))

        def body(i, carry):
            for k in range(UNROLL):
                step(i * (UNROLL * LANES) + k * LANES)
            return carry

        lax.fori_loop(0, nsteps, body, 0)

        obase = pl.multiple_of(wid * NP, 128)
        st0 = pltpu.async_copy(ms_v, msum_hbm.at[pl.ds(obase, NP)], sem0)
        st1 = pltpu.async_copy(cn_v, cnt_hbm.at[pl.ds(obase, NP)], sem1)
        st0.wait()
        st1.wait()

    return sc_kernel(orc_p, ei)


def _tc_body(orc_ref, ms_ref, cn_ref, w1_ref, b1_ref, w2_ref, b2_ref,
             g_ref, be_ref, out_ref):
    f32 = jnp.float32
    msum = jnp.sum(ms_ref[...], axis=0, keepdims=True)
    cnt = jnp.sum(cn_ref[...], axis=0, keepdims=True)
    nm = jnp.where(cnt > 0, msum / jnp.where(cnt > 0, cnt, 1.0), 0.0)
    orc = orc_ref[...]
    scale = 1.0 / (2.0 + 1e-8)
    no = jnp.clip((orc + 1.0) * scale, 0.0, 1.0)
    nn = jnp.clip((nm + 1.0) * scale, 0.0, 1.0)

    j = lax.broadcasted_iota(jnp.int32, (DC, 1), 0)
    base = jnp.where(j < DC // 2, no, nn)
    freq = (((j % (DC // 2)) // 2) + 1).astype(f32) * math.pi
    ang = base * freq
    phi = jnp.where(j % 2 == 0, jnp.sin(ang), jnp.cos(ang))

    h = lax.dot_general(w1_ref[...], phi, (((1,), (0,)), ((), ())),
                        preferred_element_type=f32) + b1_ref[...]
    h = jnp.maximum(h, 0.0)
    h2 = lax.dot_general(w2_ref[...], h, (((1,), (0,)), ((), ())),
                         preferred_element_type=f32) + b2_ref[...]
    mu = jnp.mean(h2, axis=0, keepdims=True)
    d = h2 - mu
    var = jnp.mean(d * d, axis=0, keepdims=True)
    ln = d / jnp.sqrt(var + 1e-5) * g_ref[...] + be_ref[...]
    outT = ln + phi
    eye = (lax.broadcasted_iota(jnp.int32, (DC, DC), 0)
           == lax.broadcasted_iota(jnp.int32, (DC, DC), 1)).astype(f32)
    out_ref[...] = lax.dot_general(outT, eye, (((0,), (0,)), ((), ())),
                                   preferred_element_type=f32)


def _tc_dense(orc_row, msum_p, cnt_p, W1, b1, W2, b2, gamma, beta,
              interpret=False):
    C = 2048
    return pl.pallas_call(
        _tc_body,
        grid=(NP // C,),
        in_specs=[
            pl.BlockSpec((1, C), lambda i: (0, i)),
            pl.BlockSpec((NW, C), lambda i: (0, i)),
            pl.BlockSpec((NW, C), lambda i: (0, i)),
            pl.BlockSpec((HID, DC), lambda i: (0, 0)),
            pl.BlockSpec((HID, 1), lambda i: (0, 0)),
            pl.BlockSpec((DC, HID), lambda i: (0, 0)),
            pl.BlockSpec((DC, 1), lambda i: (0, 0)),
            pl.BlockSpec((DC, 1), lambda i: (0, 0)),
            pl.BlockSpec((DC, 1), lambda i: (0, 0)),
        ],
        out_specs=pl.BlockSpec((C, DC), lambda i: (i, 0)),
        out_shape=jax.ShapeDtypeStruct((N, DC), jnp.float32),
        interpret=interpret,
    )(orc_row, msum_p, cnt_p, W1, b1, W2, b2, gamma, beta)


def kernel(node_orc, edge_index, W1, b1, W2, b2, gamma, beta):
    orc_p = jnp.pad(node_orc, (0, NP - N))
    msum_f, cnt_f = _sc_scatter(orc_p, edge_index)
    msum_p = msum_f.reshape(NW, NP)
    cnt_p = cnt_f.reshape(NW, NP)
    return _tc_dense(node_orc.reshape(1, N), msum_p, cnt_p,
                     W1, b1.reshape(HID, 1), W2, b2.reshape(DC, 1),
                     gamma.reshape(DC, 1), beta.reshape(DC, 1))


# speedup vs baseline: 1.0706x; 1.0364x over previous
import functools
import math

import jax
import jax.numpy as jnp
from jax import lax
from jax.experimental import pallas as pl
from jax.experimental.pallas import tpu as pltpu
from jax.experimental.pallas import tpu_sc as plsc

N = 10000
E = 160000
DC = 16
HID = 32

NC = 2
NS = 16
NW = NC * NS
LANES = 16

NP = 10240
CBIG = 5120
CSML = 4992


def _sc_scatter(orc_p, ei):
    mesh = plsc.VectorSubcoreMesh(core_axis_name="c", subcore_axis_name="s")

    @functools.partial(
        pl.kernel,
        out_type=(
            jax.ShapeDtypeStruct((NW * NP,), jnp.float32),
            jax.ShapeDtypeStruct((NW * NP,), jnp.float32),
        ),
        mesh=mesh,
        compiler_params=pltpu.CompilerParams(
            use_tc_tiling_on_sc=True, needs_layout_passes=False),
        scratch_types=(
            pltpu.VMEM((N,), jnp.float32),
            pltpu.VMEM((2, CBIG), jnp.int32),
            pltpu.VMEM((NP,), jnp.float32),
            pltpu.VMEM((NP,), jnp.float32),
            pltpu.SemaphoreType.DMA,
            pltpu.SemaphoreType.DMA,
        ),
    )
    def sc_kernel(orc_hbm, ei_hbm, msum_hbm, cnt_hbm,
                  orc_v, uv_v, ms_v, cn_v, sem0, sem1):
        wid = lax.axis_index("s") * NC + lax.axis_index("c")
        big = wid < 2
        base = pl.multiple_of(
            jnp.where(big, wid * CBIG, 2 * CBIG + (wid - 2) * CSML), 128)
        cp0 = pltpu.async_copy(orc_hbm, orc_v, sem0)

        @pl.when(big)
        def _():
            cp1 = pltpu.async_copy(ei_hbm.at[:, pl.ds(base, CBIG)],
                                   uv_v.at[:, pl.ds(0, CBIG)], sem1)
            cp1.wait()

        @pl.when(jnp.logical_not(big))
        def _():
            cp1 = pltpu.async_copy(ei_hbm.at[:, pl.ds(base, CSML)],
                                   uv_v.at[:, pl.ds(0, CSML)], sem1)
            cp1.wait()

        zeros = jnp.zeros((LANES,), jnp.float32)
        ZUNROLL = 8

        def zero_body(i, carry):
            for k in range(ZUNROLL):
                off = (i * ZUNROLL + k) * LANES
                ms_v[pl.ds(off, LANES)] = zeros
                cn_v[pl.ds(off, LANES)] = zeros
            return carry

        lax.fori_loop(0, NP // (LANES * ZUNROLL), zero_body, 0)
        cp0.wait()

        ones = jnp.ones((LANES,), jnp.float32)

        def step(off):
            u16 = uv_v[0, pl.ds(off, LANES)]
            v16 = uv_v[1, pl.ds(off, LANES)]
            ou = plsc.load_gather(orc_v, [u16])
            ov = plsc.load_gather(orc_v, [v16])
            plsc.addupdate_scatter(ms_v, [u16], ov)
            plsc.addupdate_scatter(ms_v, [v16], ou)
            plsc.addupdate_scatter(cn_v, [u16], ones)
            plsc.addupdate_scatter(cn_v, [v16], ones)

        UNROLL = 8
        nsteps = jnp.where(big, CBIG // (UNROLL * LANES),
                           CSML // (UNROLL * LANES))

        def body(i, carry):
            for k in range(UNROLL):
                step(i * (UNROLL * LANES) + k * LANES)
            return carry

        lax.fori_loop(0, nsteps, body, 0)

        obase = pl.multiple_of(wid * NP, 128)
        st0 = pltpu.async_copy(ms_v, msum_hbm.at[pl.ds(obase, NP)], sem0)
        st1 = pltpu.async_copy(cn_v, cnt_hbm.at[pl.ds(obase, NP)], sem1)
        st0.wait()
        st1.wait()

    return sc_kernel(orc_p, ei)


def _tc_body(orc_ref, *refs):
    f32 = jnp.float32
    ms_refs = refs[:NW]
    cn_refs = refs[NW:2 * NW]
    w1_ref, b1_ref, w2_ref, b2_ref, g_ref, be_ref, out_ref = refs[2 * NW:]

    def tree_sum(vals):
        while len(vals) > 1:
            vals = [a + b for a, b in zip(vals[::2], vals[1::2])]
        return vals[0]

    msum = tree_sum([r[...] for r in ms_refs]).reshape(1, -1)
    cnt = tree_sum([r[...] for r in cn_refs]).reshape(1, -1)
    nm = jnp.where(cnt > 0, msum / jnp.where(cnt > 0, cnt, 1.0), 0.0)
    orc = orc_ref[...]
    scale = 1.0 / (2.0 + 1e-8)
    no = jnp.clip((orc + 1.0) * scale, 0.0, 1.0)
    nn = jnp.clip((nm + 1.0) * scale, 0.0, 1.0)

    j = lax.broadcasted_iota(jnp.int32, (DC, 1), 0)
    base = jnp.where(j < DC // 2, no, nn)
    freq = (((j % (DC // 2)) // 2) + 1).astype(f32) * math.pi
    ang = base * freq
    phi = jnp.where(j % 2 == 0, jnp.sin(ang), jnp.cos(ang))

    h = lax.dot_general(w1_ref[...], phi, (((1,), (0,)), ((), ())),
                        preferred_element_type=f32) + b1_ref[...]
    h = jnp.maximum(h, 0.0)
    h2 = lax.dot_general(w2_ref[...], h, (((1,), (0,)), ((), ())),
                         preferred_element_type=f32) + b2_ref[...]
    mu = jnp.mean(h2, axis=0, keepdims=True)
    d = h2 - mu
    var = jnp.mean(d * d, axis=0, keepdims=True)
    ln = d / jnp.sqrt(var + 1e-5) * g_ref[...] + be_ref[...]
    outT = ln + phi
    eye = (lax.broadcasted_iota(jnp.int32, (DC, DC), 0)
           == lax.broadcasted_iota(jnp.int32, (DC, DC), 1)).astype(f32)
    out_ref[...] = lax.dot_general(outT, eye, (((0,), (0,)), ((), ())),
                                   preferred_element_type=f32)


def _tc_dense(orc_row, msum_f, cnt_f, W1, b1, W2, b2, gamma, beta,
              interpret=False):
    C = 2048
    nb = NP // C

    def row_spec(w):
        return pl.BlockSpec((C,), lambda i, w=w: (w * nb + i,))

    return pl.pallas_call(
        _tc_body,
        grid=(nb,),
        in_specs=(
            [pl.BlockSpec((1, C), lambda i: (0, i))]
            + [row_spec(w) for w in range(NW)]
            + [row_spec(w) for w in range(NW)]
            + [
                pl.BlockSpec((HID, DC), lambda i: (0, 0)),
                pl.BlockSpec((HID, 1), lambda i: (0, 0)),
                pl.BlockSpec((DC, HID), lambda i: (0, 0)),
                pl.BlockSpec((DC, 1), lambda i: (0, 0)),
                pl.BlockSpec((DC, 1), lambda i: (0, 0)),
                pl.BlockSpec((DC, 1), lambda i: (0, 0)),
            ]
        ),
        out_specs=pl.BlockSpec((C, DC), lambda i: (i, 0)),
        out_shape=jax.ShapeDtypeStruct((N, DC), jnp.float32),
        interpret=interpret,
    )(orc_row, *([msum_f] * NW), *([cnt_f] * NW),
      W1, b1, W2, b2, gamma, beta)


def kernel(node_orc, edge_index, W1, b1, W2, b2, gamma, beta):
    msum_f, cnt_f = _sc_scatter(node_orc, edge_index)
    return _tc_dense(node_orc.reshape(1, N), msum_f, cnt_f,
                     W1, b1.reshape(HID, 1), W2, b2.reshape(DC, 1),
                     gamma.reshape(DC, 1), beta.reshape(DC, 1))
